# baseline (speedup 1.0000x reference)
import functools

import jax
import jax.numpy as jnp
from jax import lax
from jax.experimental import pallas as pl
from jax.experimental.pallas import tpu as pltpu
from jax.experimental.pallas import tpu_sc as plsc

B, L = 16384, 50
TOKEN_DIM = 64
EMBED_DIM = 256
EPS = 1e-5

NC, NS = 2, 16
NW = NC * NS
BPW = B // NW
PBLK = 128
NP = BPW // PBLK
NCHUNK = NP * L
NBUF = 4
NQ = TOKEN_DIM // 16


@functools.lru_cache(maxsize=1)
def _make_gather_pool():
    mesh = plsc.VectorSubcoreMesh(core_axis_name="c", subcore_axis_name="s",
                                  num_cores=NC, num_subcores=NS)
    return pl.kernel(
        _gather_pool_body,
        mesh=mesh,
        out_type=jax.ShapeDtypeStruct((B, TOKEN_DIM), jnp.float32),
        scratch_types=(
            [pltpu.VMEM((NCHUNK, PBLK), jnp.int32)]
            + [pltpu.VMEM((PBLK, TOKEN_DIM), jnp.float32)
               for _ in range(NBUF)]
            + [pltpu.VMEM((BPW, TOKEN_DIM), jnp.float32)]
            + [pltpu.SemaphoreType.DMA for _ in range(NBUF)]
        ),
        compiler_params=pltpu.CompilerParams(use_tc_tiling_on_sc=False),
    )


def _gather_pool_body(tokT_hbm, table_hbm, out_hbm, *refs):
    ids_v = refs[0]
    bufs = refs[1:1 + NBUF]
    pooled_v = refs[1 + NBUF]
    sems = refs[2 + NBUF:2 + 2 * NBUF]

    wid = lax.axis_index("s") * NC + lax.axis_index("c")
    base_b = wid * BPW

    for p in range(NP):
        pltpu.sync_copy(tokT_hbm.at[:, pl.ds(base_b + p * PBLK, PBLK)],
                        ids_v.at[pl.ds(p * L, L)])

    zeros = jnp.zeros((16,), jnp.float32)

    def zero_body(r, carry):
        for q in range(NQ):
            pooled_v[r, pl.ds(q * 16, 16)] = zeros
        return carry

    lax.fori_loop(0, BPW, zero_body, 0)

    for s in range(NBUF):
        pltpu.async_copy(table_hbm.at[ids_v.at[s]], bufs[s], sems[s])

    def outer(i, carry):
        for s in range(NBUF):
            m = NBUF * i + s
            p = m // L
            row0 = p * PBLK
            pltpu.make_async_copy(table_hbm.at[ids_v.at[m]], bufs[s],
                                  sems[s]).wait()
            buf = bufs[s]

            def acc_body(r4, carry2, _buf=buf, _row0=row0):
                for u in range(4):
                    r = 4 * r4 + u
                    for q in range(NQ):
                        plsc.addupdate(
                            pooled_v.at[_row0 + r, pl.ds(q * 16, 16)],
                            _buf[r, pl.ds(q * 16, 16)])
                return carry2

            lax.fori_loop(0, PBLK // 4, acc_body, 0)

            @pl.when(m + NBUF < NCHUNK)
            def _():
                pltpu.async_copy(table_hbm.at[ids_v.at[m + NBUF]], bufs[s],
                                 sems[s])
        return carry

    lax.fori_loop(0, NCHUNK // NBUF, outer, 0)
    pltpu.sync_copy(pooled_v, out_hbm.at[pl.ds(base_b, BPW)])


def _head_body(x_ref, w_ref, b_ref, g_ref, bt_ref, o_ref):
    x = x_ref[...] * (1.0 / L)
    h = lax.dot_general(x, w_ref[...], (((1,), (1,)), ((), ())),
                        precision=lax.Precision.HIGHEST,
                        preferred_element_type=jnp.float32)
    h = h + b_ref[...]
    mu = jnp.mean(h, axis=-1, keepdims=True)
    d = h - mu
    var = jnp.mean(d * d, axis=-1, keepdims=True)
    o_ref[...] = d * lax.rsqrt(var + EPS) * g_ref[...] + bt_ref[...]


def kernel(token_ids, table, W, b, gamma, beta):
    tok = token_ids if token_ids.dtype == jnp.int32 else (
        token_ids.astype(jnp.int32))
    pooled_sum = _make_gather_pool()(tok.T, table)

    BS = 1024
    out = pl.pallas_call(
        _head_body,
        grid=(B // BS,),
        in_specs=[
            pl.BlockSpec((BS, TOKEN_DIM), lambda i: (i, 0)),
            pl.BlockSpec((EMBED_DIM, TOKEN_DIM), lambda i: (0, 0)),
            pl.BlockSpec((1, EMBED_DIM), lambda i: (0, 0)),
            pl.BlockSpec((1, EMBED_DIM), lambda i: (0, 0)),
            pl.BlockSpec((1, EMBED_DIM), lambda i: (0, 0)),
        ],
        out_specs=pl.BlockSpec((BS, EMBED_DIM), lambda i: (i, 0)),
        out_shape=jax.ShapeDtypeStruct((B, EMBED_DIM), jnp.float32),
    )(pooled_sum, W, b.reshape(1, EMBED_DIM), gamma.reshape(1, EMBED_DIM),
      beta.reshape(1, EMBED_DIM))
    return out


# speedup vs baseline: 1.4438x; 1.4438x over previous
import functools

import jax
import jax.numpy as jnp
from jax import lax
from jax.experimental import pallas as pl
from jax.experimental.layout import Format, Layout, with_layout_constraint
from jax.experimental.pallas import tpu as pltpu
from jax.experimental.pallas import tpu_sc as plsc

B, L = 16384, 50
LP = 64
TOKEN_DIM = 64
EMBED_DIM = 256
EPS = 1e-5

NC, NS = 2, 16
NW = NC * NS
BPW = B // NW
CHUNK_B = 2
NCHUNK = BPW // CHUNK_B
NBUF = 4


@functools.lru_cache(maxsize=1)
def _make_gather_pool():
    mesh = plsc.VectorSubcoreMesh(core_axis_name="c", subcore_axis_name="s",
                                  num_cores=NC, num_subcores=NS)
    return pl.kernel(
        _gather_pool_body,
        mesh=mesh,
        out_type=jax.ShapeDtypeStruct((B, TOKEN_DIM), jnp.float32),
        scratch_types=(
            [pltpu.VMEM((BPW, L), jnp.int32),
             pltpu.VMEM((NCHUNK, 128), jnp.int32)]
            + [pltpu.VMEM((128, TOKEN_DIM), jnp.float32) for _ in range(NBUF)]
            + [pltpu.VMEM((BPW, TOKEN_DIM), jnp.float32)]
            + [pltpu.SemaphoreType.DMA for _ in range(NBUF)]
        ),
        compiler_params=pltpu.CompilerParams(use_tc_tiling_on_sc=False),
    )


def _gather_pool_body(tok_hbm, table_hbm, out_hbm, *refs):
    ids_raw = refs[0]
    ids2d = refs[1]
    bufs = refs[2:2 + NBUF]
    pooled_v = refs[2 + NBUF]
    sems = refs[3 + NBUF:3 + 2 * NBUF]

    wid = lax.axis_index("s") * NC + lax.axis_index("c")
    pltpu.sync_copy(tok_hbm.at[pl.ds(wid * BPW, BPW)], ids_raw)

    def pack_body(c, carry):
        for half in range(2):
            bb = 2 * c + half
            dst0 = half * LP
            for k in range(3):
                v = ids_raw[bb, pl.ds(16 * k, 16)]
                ids2d[c, pl.ds(dst0 + 16 * k, 16)] = v + v
            tail = ids_raw[bb, pl.ds(34, 16)]
            tail = tail + tail
            ids2d[c, pl.ds(dst0 + 48, 16)] = tail
            ids2d[c, pl.ds(dst0 + 34, 16)] = tail
        return carry

    lax.fori_loop(0, NCHUNK, pack_body, 0)

    for s in range(NBUF):
        pltpu.async_copy(table_hbm.at[ids2d.at[s]], bufs[s], sems[s])

    NQ = TOKEN_DIM // 16

    def outer(i, carry):
        for s in range(NBUF):
            c = NBUF * i + s
            pltpu.make_async_copy(table_hbm.at[ids2d.at[c]], bufs[s],
                                  sems[s]).wait()
            buf = bufs[s]
            for sub in range(CHUNK_B):
                def acc_body(r, acc, _sub=sub, _buf=buf):
                    base = _sub * LP + 2 * r
                    return tuple(
                        acc[q] + (_buf[base, pl.ds(q * 16, 16)]
                                  + _buf[base + 1, pl.ds(q * 16, 16)])
                        for q in range(NQ))

                acc = lax.fori_loop(
                    0, L // 2, acc_body,
                    tuple(jnp.zeros((16,), jnp.float32) for _ in range(NQ)))
                row = c * CHUNK_B + sub
                for q in range(NQ):
                    pooled_v[row, pl.ds(q * 16, 16)] = acc[q]

            @pl.when(c + NBUF < NCHUNK)
            def _():
                pltpu.async_copy(table_hbm.at[ids2d.at[c + NBUF]], bufs[s],
                                 sems[s])
        return carry

    lax.fori_loop(0, NCHUNK // NBUF, outer, 0)
    pltpu.sync_copy(pooled_v, out_hbm.at[pl.ds(wid * BPW, BPW)])


def _head_body(x_ref, w_ref, b_ref, g_ref, bt_ref, o_ref):
    x = x_ref[...] * (1.0 / L)
    h = lax.dot_general(x, w_ref[...], (((1,), (1,)), ((), ())),
                        precision=lax.Precision.HIGHEST,
                        preferred_element_type=jnp.float32)
    h = h + b_ref[...]
    mu = jnp.mean(h, axis=-1, keepdims=True)
    d = h - mu
    var = jnp.mean(d * d, axis=-1, keepdims=True)
    o_ref[...] = d * lax.rsqrt(var + EPS) * g_ref[...] + bt_ref[...]


def kernel(token_ids, table, W, b, gamma, beta):
    tok = token_ids if token_ids.dtype == jnp.int32 else (
        token_ids.astype(jnp.int32))
    table_lin = jnp.pad(table, ((0, 0), (0, 64))).reshape(2000000, TOKEN_DIM)
    pooled_sum = _make_gather_pool()(tok, table_lin)

    BS = 1024
    out = pl.pallas_call(
        _head_body,
        grid=(B // BS,),
        in_specs=[
            pl.BlockSpec((BS, TOKEN_DIM), lambda i: (i, 0)),
            pl.BlockSpec((EMBED_DIM, TOKEN_DIM), lambda i: (0, 0)),
            pl.BlockSpec((1, EMBED_DIM), lambda i: (0, 0)),
            pl.BlockSpec((1, EMBED_DIM), lambda i: (0, 0)),
            pl.BlockSpec((1, EMBED_DIM), lambda i: (0, 0)),
        ],
        out_specs=pl.BlockSpec((BS, EMBED_DIM), lambda i: (i, 0)),
        out_shape=jax.ShapeDtypeStruct((B, EMBED_DIM), jnp.float32),
    )(pooled_sum, W, b.reshape(1, EMBED_DIM), gamma.reshape(1, EMBED_DIM),
      beta.reshape(1, EMBED_DIM))
    return out
